# baseline (speedup 1.0000x reference)
import functools

import jax
import jax.numpy as jnp
from jax import lax
from jax.experimental import pallas as pl
from jax.experimental.pallas import tpu as pltpu
from jax.experimental.pallas import tpu_sc as plsc

N_NODES = 10000
N_EDGES = 160000
D_IN = 256
D_OUT = 256
DH = D_IN // 2

NC = 2
NS = 16
CH = 80
G = 5
NG = N_EDGES // (G * CH * NS)
T = NG * G
R = 4
AHEAD = 2
NPT = 632
LAST_START = N_NODES - NPT


def _sc_aggregate(x, e4, zrows):
    mesh = plsc.VectorSubcoreMesh(core_axis_name="c", subcore_axis_name="s")

    @functools.partial(
        pl.kernel,
        out_type=jax.ShapeDtypeStruct((NC, N_NODES, DH), jnp.float32),
        mesh=mesh,
        scratch_types=[
            pltpu.VMEM_SHARED((N_NODES, DH), jnp.float32),
            pltpu.VMEM((3, 2, G, CH), jnp.int32),
            pltpu.VMEM((R, CH, DH), jnp.float32),
            pltpu.SemaphoreType.DMA,
            pltpu.SemaphoreType.DMA,
            pltpu.SemaphoreType.DMA,
        ],
    )
    def body(x_hbm, e_hbm, z_hbm, out_hbm, acc, ibuf, rows, sem_g, sem_s,
             sem_i):
        c = lax.axis_index("c")
        s = lax.axis_index("s")

        def gather(j):
            return pltpu.make_async_copy(
                x_hbm.at[ibuf.at[(j // G) % 3, 0, j % G], pl.ds(c * DH, DH)],
                rows.at[j % R], sem_g)

        def scatter(j):
            return pltpu.make_async_copy(
                rows.at[j % R],
                acc.at[ibuf.at[(j // G) % 3, 1, j % G]], sem_s)

        def stage(g):
            return pltpu.make_async_copy(e_hbm.at[s * NG + g], ibuf.at[g % 3],
                                         sem_i)

        stage(0).start()
        stage(0).wait()
        stage(1).start()
        for j0 in range(AHEAD):
            gather(j0).start()
        start = pl.multiple_of(jnp.minimum(s * NPT, LAST_START), 8)
        pltpu.sync_copy(z_hbm, acc.at[pl.ds(start, NPT)])
        plsc.subcore_barrier()

        def step(j, _):
            @pl.when(j >= AHEAD)
            def _():
                scatter(j - AHEAD).wait()

            @pl.when(j + AHEAD < T)
            def _():
                jn = j + AHEAD

                @pl.when(jn % G == 0)
                def _():
                    g2 = jn // G
                    stage(g2).wait()

                    @pl.when(g2 + 1 < NG)
                    def _():
                        stage(g2 + 1).start()
                gather(jn).start()

            gather(j).wait()
            pltpu.async_copy(rows.at[j % R],
                             acc.at[ibuf.at[(j // G) % 3, 1, j % G]],
                             sem_s, add=True)
            return 0

        lax.fori_loop(0, T, step, 0, unroll=False)
        for j0 in range(T - AHEAD, T):
            scatter(j0).wait()
        plsc.subcore_barrier()

        pltpu.sync_copy(
            acc.at[pl.ds(start, NPT)],
            out_hbm.at[c].at[pl.ds(start, NPT)],
        )

    return body(x, e4, zrows)


def _tc_linear(agg2, W, b2):
    BN = 1000
    dn = (((1,), (1,)), ((), ()))

    def mm(a_ref, w_ref, b_ref, o_ref):
        acc = lax.dot_general(a_ref[0], w_ref[:, :DH], dn,
                              preferred_element_type=jnp.float32)
        acc = acc + lax.dot_general(a_ref[1], w_ref[:, DH:], dn,
                                    preferred_element_type=jnp.float32)
        o_ref[...] = acc + b_ref[...]

    return pl.pallas_call(
        mm,
        grid=(N_NODES // BN,),
        in_specs=[
            pl.BlockSpec((2, BN, DH), lambda i: (0, i, 0)),
            pl.BlockSpec((D_OUT, D_IN), lambda i: (0, 0)),
            pl.BlockSpec((1, D_OUT), lambda i: (0, 0)),
        ],
        out_specs=pl.BlockSpec((BN, D_OUT), lambda i: (i, 0)),
        out_shape=jax.ShapeDtypeStruct((N_NODES, D_OUT), jnp.float32),
    )(agg2, W, b2)


def kernel(x, edge_index, W, b):
    e4 = (edge_index.astype(jnp.int32)
          .reshape(2, NS * NG, G, CH).transpose(1, 0, 2, 3))
    zrows = jnp.zeros((NPT, DH), jnp.float32)
    b2 = b.reshape(1, D_OUT)
    agg2 = _sc_aggregate(x, e4, zrows)
    return _tc_linear(agg2, W, b2)


# speedup vs baseline: 1.0219x; 1.0219x over previous
import functools

import jax
import jax.numpy as jnp
from jax import lax
from jax.experimental import pallas as pl
from jax.experimental.pallas import tpu as pltpu
from jax.experimental.pallas import tpu_sc as plsc

N_NODES = 10000
N_EDGES = 160000
D_IN = 256
D_OUT = 256
DH = D_IN // 2

NC = 2
NS = 16
CH = 80
G = 5
NG = N_EDGES // (G * CH * NS)
T = NG * G
R = 4
AHEAD = 2
NPT = 632
LAST_START = N_NODES - NPT


def _sc_aggregate(x, e4, zrows):
    mesh = plsc.VectorSubcoreMesh(core_axis_name="c", subcore_axis_name="s")

    @functools.partial(
        pl.kernel,
        out_type=jax.ShapeDtypeStruct((NC, N_NODES, DH), jnp.float32),
        mesh=mesh,
        scratch_types=[
            pltpu.VMEM_SHARED((N_NODES, DH), jnp.float32),
            pltpu.VMEM((3, 2, G, CH), jnp.int32),
            pltpu.VMEM((R, CH, DH), jnp.float32),
            pltpu.SemaphoreType.DMA,
            pltpu.SemaphoreType.DMA,
            pltpu.SemaphoreType.DMA,
        ],
    )
    def body(x_hbm, e_hbm, z_hbm, out_hbm, acc, ibuf, rows, sem_g, sem_s,
             sem_i):
        c = lax.axis_index("c")
        s = lax.axis_index("s")

        def gather(j):
            return pltpu.make_async_copy(
                x_hbm.at[ibuf.at[(j // G) % 3, 0, j % G], pl.ds(c * DH, DH)],
                rows.at[j % R], sem_g)

        def scatter(j):
            return pltpu.make_async_copy(
                rows.at[j % R],
                acc.at[ibuf.at[(j // G) % 3, 1, j % G]], sem_s)

        def stage(g):
            return pltpu.make_async_copy(e_hbm.at[s * NG + g], ibuf.at[g % 3],
                                         sem_i)

        stage(0).start()
        stage(0).wait()
        stage(1).start()
        for j0 in range(AHEAD):
            gather(j0).start()
        start = pl.multiple_of(jnp.minimum(s * NPT, LAST_START), 8)
        pltpu.sync_copy(z_hbm, acc.at[pl.ds(start, NPT)])
        plsc.subcore_barrier()

        def step(j, _):
            @pl.when(j >= AHEAD)
            def _():
                scatter(j - AHEAD).wait()

            @pl.when(j + AHEAD < T)
            def _():
                jn = j + AHEAD

                @pl.when(jn % G == 0)
                def _():
                    g2 = jn // G
                    stage(g2).wait()

                    @pl.when(g2 + 1 < NG)
                    def _():
                        stage(g2 + 1).start()
                gather(jn).start()

            gather(j).wait()
            pltpu.async_copy(rows.at[j % R],
                             acc.at[ibuf.at[(j // G) % 3, 1, j % G]],
                             sem_s, add=True)
            return 0

        lax.fori_loop(0, T, step, 0, unroll=False)
        for j0 in range(T - AHEAD, T):
            scatter(j0).wait()
        plsc.subcore_barrier()

        pltpu.sync_copy(
            acc.at[pl.ds(start, NPT)],
            out_hbm.at[c].at[pl.ds(start, NPT)],
        )

    return body(x, e4, zrows)


def _tc_linear(agg2, W, b2):
    BN = 2000
    dn = (((1,), (1,)), ((), ()))

    def mm(a_ref, w_ref, b_ref, o_ref):
        acc = lax.dot_general(a_ref[0], w_ref[:, :DH], dn,
                              preferred_element_type=jnp.float32)
        acc = acc + lax.dot_general(a_ref[1], w_ref[:, DH:], dn,
                                    preferred_element_type=jnp.float32)
        o_ref[...] = acc + b_ref[...]

    return pl.pallas_call(
        mm,
        grid=(N_NODES // BN,),
        in_specs=[
            pl.BlockSpec((2, BN, DH), lambda i: (0, i, 0)),
            pl.BlockSpec((D_OUT, D_IN), lambda i: (0, 0)),
            pl.BlockSpec((1, D_OUT), lambda i: (0, 0)),
        ],
        out_specs=pl.BlockSpec((BN, D_OUT), lambda i: (i, 0)),
        out_shape=jax.ShapeDtypeStruct((N_NODES, D_OUT), jnp.float32),
    )(agg2, W, b2)


def kernel(x, edge_index, W, b):
    e4 = (edge_index.astype(jnp.int32)
          .reshape(2, NS * NG, G, CH).transpose(1, 0, 2, 3))
    zrows = jnp.zeros((NPT, DH), jnp.float32)
    b2 = b.reshape(1, D_OUT)
    agg2 = _sc_aggregate(x, e4, zrows)
    return _tc_linear(agg2, W, b2)
